# Initial kernel scaffold; baseline (speedup 1.0000x reference)
import jax
import jax.numpy as jnp
from jax.experimental import pallas as pl


def kernel(input_words, output_words, noise_words, in_embeddings, out_embeddings):
    raise NotImplementedError("write your pallas kernel here")


# speedup vs baseline: 3.4601x; 3.4601x over previous
import functools

import jax
import jax.numpy as jnp
from jax import lax
from jax.experimental import pallas as pl
from jax.experimental.pallas import tpu as pltpu
from jax.experimental.pallas import tpu_sc as plsc

_B = 16384
_NS = 20
_D = 128
_C = 128
_NBUF = 4


@functools.cache
def _build_gather_kernel():
  info = plsc.get_sparse_core_info()
  nc, nsub = info.num_cores, info.num_subcores
  nw = nc * nsub

  n_small = _B // nw
  n_noise = (_B * _NS) // nw
  assert n_small % (_C * _NBUF) == 0 and n_noise % (_C * _NBUF) == 0

  mesh = plsc.VectorSubcoreMesh(core_axis_name="c", subcore_axis_name="s")

  @functools.partial(
      pl.kernel,
      out_type=[
          jax.ShapeDtypeStruct((_B, _D), jnp.float32),
          jax.ShapeDtypeStruct((_B, _D), jnp.float32),
          jax.ShapeDtypeStruct((_B * _NS, _D), jnp.float32),
      ],
      mesh=mesh,
      scratch_types=(
          [pltpu.VMEM((_C,), jnp.int32) for _ in range(_NBUF)]
          + [pltpu.VMEM((_C, _D), jnp.float32) for _ in range(_NBUF)]
          + [pltpu.SemaphoreType.DMA for _ in range(2 * _NBUF)]
      ),
  )
  def gather_kernel(iw_hbm, ow_hbm, nw_hbm, ie_hbm, oe_hbm, o1, o2, o3, *scr):
    idxb = scr[:_NBUF]
    rows = scr[_NBUF:2 * _NBUF]
    gsem = scr[2 * _NBUF:3 * _NBUF]
    wsem = scr[3 * _NBUF:4 * _NBUF]
    wid = lax.axis_index("s") * nc + lax.axis_index("c")

    def phase(idx_hbm, tab_hbm, out_hbm, rows_per_worker):
      nsteps = rows_per_worker // (_C * _NBUF)
      base = wid * rows_per_worker
      for b in range(_NBUF):
        pltpu.sync_copy(idx_hbm.at[pl.ds(base + b * _C, _C)], idxb[b])
        pltpu.async_copy(tab_hbm.at[idxb[b]], rows[b], gsem[b])

      def step(s, carry):
        for b in range(_NBUF):
          g = s * _NBUF + b
          row0 = base + g * _C
          out_slice = out_hbm.at[pl.ds(row0, _C)]
          pltpu.make_async_copy(tab_hbm.at[idxb[b]], rows[b], gsem[b]).wait()
          pltpu.async_copy(rows[b], out_slice, wsem[b])

          @pl.when(s < nsteps - 1)
          def _():
            pltpu.make_async_copy(rows[b], out_slice, wsem[b]).wait()
            pltpu.sync_copy(idx_hbm.at[pl.ds(row0 + _NBUF * _C, _C)], idxb[b])
            pltpu.async_copy(tab_hbm.at[idxb[b]], rows[b], gsem[b])

        return carry

      lax.fori_loop(0, nsteps, step, 0)
      for b in range(_NBUF):
        pltpu.make_async_copy(rows[b], out_hbm.at[pl.ds(base, _C)], wsem[b]).wait()

    phase(iw_hbm, ie_hbm, o1, n_small)
    phase(ow_hbm, oe_hbm, o2, n_small)
    phase(nw_hbm, oe_hbm, o3, n_noise)

  return gather_kernel


def kernel(input_words, output_words, noise_words, in_embeddings, out_embeddings):
  gather = _build_gather_kernel()
  o1, o2, o3 = gather(
      input_words.astype(jnp.int32),
      output_words.astype(jnp.int32),
      noise_words.astype(jnp.int32),
      in_embeddings,
      out_embeddings,
  )
  return (o1, o2, o3.reshape(_B, _NS, _D))


# speedup vs baseline: 3.4888x; 1.0083x over previous
import functools

import jax
import jax.numpy as jnp
from jax import lax
from jax.experimental import pallas as pl
from jax.experimental.pallas import tpu as pltpu
from jax.experimental.pallas import tpu_sc as plsc

_B = 16384
_NS = 20
_D = 128
_C = 128
_NBUF = 4


@functools.cache
def _build_gather_kernel():
  info = plsc.get_sparse_core_info()
  nc, nsub = info.num_cores, info.num_subcores
  nw = nc * nsub

  n_small = _B // nw
  n_noise = (_B * _NS) // nw
  assert n_small % (_C * _NBUF) == 0 and n_noise % (_C * _NBUF) == 0

  mesh = plsc.VectorSubcoreMesh(core_axis_name="c", subcore_axis_name="s")

  @functools.partial(
      pl.kernel,
      out_type=[
          jax.ShapeDtypeStruct((_B, _D), jnp.float32),
          jax.ShapeDtypeStruct((_B, _D), jnp.float32),
          jax.ShapeDtypeStruct((_B * _NS, _D), jnp.float32),
      ],
      mesh=mesh,
      scratch_types=(
          [pltpu.VMEM((n_noise // _C, _C), jnp.int32)]
          + [pltpu.VMEM((_C, _D), jnp.float32) for _ in range(_NBUF)]
          + [pltpu.SemaphoreType.DMA for _ in range(2 * _NBUF)]
      ),
  )
  def gather_kernel(iw_hbm, ow_hbm, nw_hbm, ie_hbm, oe_hbm, o1, o2, o3, *scr):
    idxall = scr[0]
    rows = scr[1:1 + _NBUF]
    gsem = scr[1 + _NBUF:1 + 2 * _NBUF]
    wsem = scr[1 + 2 * _NBUF:1 + 3 * _NBUF]
    wid = lax.axis_index("s") * nc + lax.axis_index("c")

    def phase(idx2_hbm, tab_hbm, out_hbm, rows_per_worker):
      nch = rows_per_worker // _C
      nsteps = nch // _NBUF
      base = wid * rows_per_worker
      pltpu.sync_copy(idx2_hbm.at[pl.ds(wid * nch, nch)],
                      idxall.at[pl.ds(0, nch)])
      for b in range(_NBUF):
        pltpu.async_copy(tab_hbm.at[idxall.at[b]], rows[b], gsem[b])

      def step(s, carry):
        for b in range(_NBUF):
          g = s * _NBUF + b
          out_slice = out_hbm.at[pl.ds(base + g * _C, _C)]
          pltpu.make_async_copy(tab_hbm.at[idxall.at[g]], rows[b], gsem[b]).wait()
          pltpu.async_copy(rows[b], out_slice, wsem[b])

          @pl.when(s < nsteps - 1)
          def _():
            pltpu.make_async_copy(rows[b], out_slice, wsem[b]).wait()
            pltpu.async_copy(tab_hbm.at[idxall.at[g + _NBUF]], rows[b], gsem[b])

        return carry

      lax.fori_loop(0, nsteps, step, 0)
      for b in range(_NBUF):
        pltpu.make_async_copy(rows[b], out_hbm.at[pl.ds(base, _C)], wsem[b]).wait()

    phase(iw_hbm, ie_hbm, o1, n_small)
    phase(ow_hbm, oe_hbm, o2, n_small)
    phase(nw_hbm, oe_hbm, o3, n_noise)

  return gather_kernel


def kernel(input_words, output_words, noise_words, in_embeddings, out_embeddings):
  gather = _build_gather_kernel()
  o1, o2, o3 = gather(
      input_words.astype(jnp.int32).reshape(-1, _C),
      output_words.astype(jnp.int32).reshape(-1, _C),
      noise_words.astype(jnp.int32).reshape(-1, _C),
      in_embeddings,
      out_embeddings,
  )
  return (o1, o2, o3.reshape(_B, _NS, _D))


# speedup vs baseline: 9.4677x; 2.7138x over previous
import functools

import jax
import jax.numpy as jnp
from jax import lax
from jax.experimental import pallas as pl
from jax.experimental.pallas import tpu as pltpu
from jax.experimental.pallas import tpu_sc as plsc

_B = 16384
_NS = 20
_D = 128
_C = 128
_NBUF = 4


@functools.cache
def _build_gather_kernel():
  info = plsc.get_sparse_core_info()
  nc, nsub = info.num_cores, info.num_subcores
  nw = nc * nsub

  n_small = _B // nw
  n_noise = (_B * _NS) // nw
  assert n_small % (_C * _NBUF) == 0 and n_noise % (_C * _NBUF) == 0

  mesh = plsc.VectorSubcoreMesh(core_axis_name="c", subcore_axis_name="s")

  @functools.partial(
      pl.kernel,
      out_type=[
          jax.ShapeDtypeStruct((_B, _D), jnp.float32),
          jax.ShapeDtypeStruct((_B, _D), jnp.float32),
          jax.ShapeDtypeStruct((_B * _NS, _D), jnp.float32),
      ],
      mesh=mesh,
      scratch_types=(
          [pltpu.VMEM((n_noise // _C, _C), jnp.int32)]
          + [pltpu.VMEM((_C, _D), jnp.float32) for _ in range(_NBUF)]
          + [pltpu.SemaphoreType.DMA for _ in range(2 * _NBUF)]
      ),
  )
  def gather_kernel(iw_hbm, ow_hbm, nw_hbm, ie_hbm, oe_hbm, o1, o2, o3, *scr):
    idxall = scr[0]
    rows = scr[1:1 + _NBUF]
    gsem = scr[1 + _NBUF:1 + 2 * _NBUF]
    wsem = scr[1 + 2 * _NBUF:1 + 3 * _NBUF]
    wid = lax.axis_index("s") * nc + lax.axis_index("c")

    def phase(idx2_hbm, tab_hbm, out_hbm, rows_per_worker):
      nch = rows_per_worker // _C
      nsteps = nch // _NBUF
      base = wid * rows_per_worker
      pltpu.sync_copy(idx2_hbm.at[pl.ds(wid * nch, nch)],
                      idxall.at[pl.ds(0, nch)])
      for b in range(_NBUF):
        pltpu.async_copy(tab_hbm.at[idxall.at[b]], rows[b], gsem[b])

      def step(s, carry):
        for b in range(_NBUF):
          g = s * _NBUF + b
          out_slice = out_hbm.at[pl.ds(base + g * _C, _C)]
          pltpu.make_async_copy(tab_hbm.at[idxall.at[g]], rows[b], gsem[b]).wait()
          pltpu.async_copy(rows[b], out_slice, wsem[b])

          @pl.when(s < nsteps - 1)
          def _():
            pltpu.make_async_copy(rows[b], out_slice, wsem[b]).wait()
            pltpu.async_copy(tab_hbm.at[idxall.at[g + _NBUF]], rows[b], gsem[b])

        return carry

      lax.fori_loop(0, nsteps, step, 0)
      for b in range(_NBUF):
        pltpu.make_async_copy(rows[b], out_hbm.at[pl.ds(base, _C)], wsem[b]).wait()

    phase(iw_hbm, ie_hbm, o1, n_small)
    phase(ow_hbm, oe_hbm, o2, n_small)
    phase(nw_hbm, oe_hbm, o3, n_noise)

  return gather_kernel


def kernel(input_words, output_words, noise_words, in_embeddings, out_embeddings):
  gather = _build_gather_kernel()
  noise_t = jnp.swapaxes(noise_words.astype(jnp.int32).reshape(_B, _NS), 0, 1)
  o1, o2, o3 = gather(
      input_words.astype(jnp.int32).reshape(-1, _C),
      output_words.astype(jnp.int32).reshape(-1, _C),
      noise_t.reshape(-1, _C),
      in_embeddings,
      out_embeddings,
  )
  return (o1, o2, jnp.swapaxes(o3.reshape(_NS, _B, _D), 0, 1))


# speedup vs baseline: 9.4843x; 1.0017x over previous
import functools

import jax
import jax.numpy as jnp
from jax import lax
from jax.experimental import pallas as pl
from jax.experimental.pallas import tpu as pltpu
from jax.experimental.pallas import tpu_sc as plsc

_B = 16384
_NS = 20
_D = 128
_C = 128
_NBUF = 5


@functools.cache
def _build_gather_kernel():
  info = plsc.get_sparse_core_info()
  nc, nsub = info.num_cores, info.num_subcores
  nw = nc * nsub

  n_small = _B // nw
  n_noise = (_B * _NS) // nw
  assert n_small % _C == 0 and n_noise % _C == 0
  assert (n_small // _C) % min(_NBUF, n_small // _C) == 0
  assert (n_noise // _C) % _NBUF == 0

  mesh = plsc.VectorSubcoreMesh(core_axis_name="c", subcore_axis_name="s")

  @functools.partial(
      pl.kernel,
      out_type=[
          jax.ShapeDtypeStruct((_B, _D), jnp.float32),
          jax.ShapeDtypeStruct((_B, _D), jnp.float32),
          jax.ShapeDtypeStruct((_B * _NS, _D), jnp.float32),
      ],
      mesh=mesh,
      scratch_types=(
          [pltpu.VMEM((n_noise // _C, _C), jnp.int32)]
          + [pltpu.VMEM((_C, _D), jnp.float32) for _ in range(_NBUF)]
          + [pltpu.SemaphoreType.DMA for _ in range(2 * _NBUF)]
      ),
  )
  def gather_kernel(iw_hbm, ow_hbm, nw_hbm, ie_hbm, oe_hbm, o1, o2, o3, *scr):
    idxall = scr[0]
    rows = scr[1:1 + _NBUF]
    gsem = scr[1 + _NBUF:1 + 2 * _NBUF]
    wsem = scr[1 + 2 * _NBUF:1 + 3 * _NBUF]
    wid = lax.axis_index("s") * nc + lax.axis_index("c")

    def phase(idx2_hbm, tab_hbm, out_hbm, rows_per_worker):
      nch = rows_per_worker // _C
      nbuf = min(_NBUF, nch)
      nsteps = nch // nbuf
      base = wid * rows_per_worker
      pltpu.sync_copy(idx2_hbm.at[pl.ds(wid * nch, nch)],
                      idxall.at[pl.ds(0, nch)])
      for b in range(nbuf):
        pltpu.async_copy(tab_hbm.at[idxall.at[b]], rows[b], gsem[b])

      def step(s, carry):
        for b in range(nbuf):
          g = s * nbuf + b
          out_slice = out_hbm.at[pl.ds(base + g * _C, _C)]
          pltpu.make_async_copy(tab_hbm.at[idxall.at[g]], rows[b], gsem[b]).wait()
          pltpu.async_copy(rows[b], out_slice, wsem[b])

          @pl.when(s < nsteps - 1)
          def _():
            pltpu.make_async_copy(rows[b], out_slice, wsem[b]).wait()
            pltpu.async_copy(tab_hbm.at[idxall.at[g + nbuf]], rows[b], gsem[b])

        return carry

      lax.fori_loop(0, nsteps, step, 0)
      for b in range(nbuf):
        pltpu.make_async_copy(rows[b], out_hbm.at[pl.ds(base, _C)], wsem[b]).wait()

    phase(iw_hbm, ie_hbm, o1, n_small)
    phase(ow_hbm, oe_hbm, o2, n_small)
    phase(nw_hbm, oe_hbm, o3, n_noise)

  return gather_kernel


def kernel(input_words, output_words, noise_words, in_embeddings, out_embeddings):
  gather = _build_gather_kernel()
  noise_t = jnp.swapaxes(noise_words.astype(jnp.int32).reshape(_B, _NS), 0, 1)
  o1, o2, o3 = gather(
      input_words.astype(jnp.int32).reshape(-1, _C),
      output_words.astype(jnp.int32).reshape(-1, _C),
      noise_t.reshape(-1, _C),
      in_embeddings,
      out_embeddings,
  )
  return (o1, o2, jnp.swapaxes(o3.reshape(_NS, _B, _D), 0, 1))


# speedup vs baseline: 10.3963x; 1.0962x over previous
import functools

import jax
import jax.numpy as jnp
from jax import lax
from jax.experimental import pallas as pl
from jax.experimental.pallas import tpu as pltpu
from jax.experimental.pallas import tpu_sc as plsc

_B = 16384
_NS = 20
_D = 128
_C = 128
_NBUF = 5
_L = 16


@functools.cache
def _build_gather_kernel():
  info = plsc.get_sparse_core_info()
  nc, nsub = info.num_cores, info.num_subcores
  nw = nc * nsub

  n_small = _B // nw
  n_noise = (_B * _NS) // nw
  nch_noise = n_noise // _C
  assert nch_noise % _NBUF == 0 and n_small % _C == 0

  mesh = plsc.VectorSubcoreMesh(core_axis_name="c", subcore_axis_name="s")

  @functools.partial(
      pl.kernel,
      out_type=[
          jax.ShapeDtypeStruct((_B, _D), jnp.float32),
          jax.ShapeDtypeStruct((_B, _D), jnp.float32),
          jax.ShapeDtypeStruct((_B * _NS, _D), jnp.float32),
      ],
      mesh=mesh,
      scratch_types=(
          [pltpu.VMEM((nch_noise, _C), jnp.int32),
           pltpu.VMEM((nch_noise, _C), jnp.int32)]
          + [pltpu.VMEM((_C, _D), jnp.float32) for _ in range(_NBUF)]
          + [pltpu.SemaphoreType.DMA for _ in range(2 * _NBUF)]
      ),
  )
  def gather_kernel(iw_hbm, ow_hbm, nw_hbm, ie_hbm, oe_hbm, o1, o2, o3, *scr):
    idxrows = scr[0]
    outrows = scr[1]
    rows = scr[2:2 + _NBUF]
    gsem = scr[2 + _NBUF:2 + 2 * _NBUF]
    wsem = scr[2 + 2 * _NBUF:2 + 3 * _NBUF]
    wid = lax.axis_index("s") * nc + lax.axis_index("c")
    lanes = lax.iota(jnp.int32, _L)

    def small_phase(idx2_hbm, tab_hbm, out_hbm):
      nch = n_small // _C
      base = wid * n_small
      pltpu.sync_copy(idx2_hbm.at[pl.ds(wid * nch, nch)],
                      idxrows.at[pl.ds(0, nch)])
      for b in range(nch):
        pltpu.async_copy(tab_hbm.at[idxrows.at[b]], rows[b], gsem[b])
      for b in range(nch):
        out_slice = out_hbm.at[pl.ds(base + b * _C, _C)]
        pltpu.make_async_copy(tab_hbm.at[idxrows.at[b]], rows[b], gsem[b]).wait()
        pltpu.async_copy(rows[b], out_slice, wsem[b])
      for b in range(nch):
        pltpu.make_async_copy(rows[b], out_hbm.at[pl.ds(base, _C)], wsem[b]).wait()

    small_phase(iw_hbm, ie_hbm, o1)
    small_phase(ow_hbm, oe_hbm, o2)

    pltpu.sync_copy(nw_hbm.at[pl.ds(wid * nch_noise, nch_noise)], idxrows)

    def orow(r, carry):
      nvec, bvec = carry
      for j in range(_C // _L):
        outrows[r, pl.ds(j * _L, _L)] = nvec * _B + bvec
        nxt = nvec + _L
        wrap = nxt >= _NS
        nvec = jnp.where(wrap, nxt - _NS, nxt)
        bvec = jnp.where(wrap, bvec + 1, bvec)
      return (nvec, bvec)

    lax.fori_loop(0, nch_noise, orow,
                  (lanes, jnp.full((_L,), wid * (_B // nw), jnp.int32)))

    for b in range(_NBUF):
      pltpu.async_copy(oe_hbm.at[idxrows.at[b]], rows[b], gsem[b])

    def step(s, carry):
      for b in range(_NBUF):
        g = s * _NBUF + b
        pltpu.make_async_copy(oe_hbm.at[idxrows.at[g]], rows[b], gsem[b]).wait()
        pltpu.async_copy(rows[b], o3.at[outrows.at[g]], wsem[b])

        @pl.when(s < (nch_noise // _NBUF) - 1)
        def _():
          pltpu.make_async_copy(rows[b], o3.at[outrows.at[g]], wsem[b]).wait()
          pltpu.async_copy(oe_hbm.at[idxrows.at[g + _NBUF]], rows[b], gsem[b])

      return carry

    lax.fori_loop(0, nch_noise // _NBUF, step, 0)
    for b in range(_NBUF):
      pltpu.make_async_copy(rows[b], o3.at[outrows.at[nch_noise - _NBUF + b]],
                            wsem[b]).wait()

  return gather_kernel


def kernel(input_words, output_words, noise_words, in_embeddings, out_embeddings):
  gather = _build_gather_kernel()
  o1, o2, o3 = gather(
      input_words.astype(jnp.int32).reshape(-1, _C),
      output_words.astype(jnp.int32).reshape(-1, _C),
      noise_words.astype(jnp.int32).reshape(-1, _C),
      in_embeddings,
      out_embeddings,
  )
  return (o1, o2, jnp.swapaxes(o3.reshape(_NS, _B, _D), 0, 1))


# speedup vs baseline: 10.4517x; 1.0053x over previous
import functools

import jax
import jax.numpy as jnp
from jax import lax
from jax.experimental import pallas as pl
from jax.experimental.pallas import tpu as pltpu
from jax.experimental.pallas import tpu_sc as plsc

_B = 16384
_NS = 20
_D = 128
_C = 128
_NBUF = 5
_L = 16


@functools.cache
def _build_gather_kernel():
  info = plsc.get_sparse_core_info()
  nc, nsub = info.num_cores, info.num_subcores
  nw = nc * nsub

  n_small = _B // nw
  n_noise = (_B * _NS) // nw
  nch_noise = n_noise // _C
  assert nch_noise % _NBUF == 0 and n_small % _C == 0

  mesh = plsc.VectorSubcoreMesh(core_axis_name="c", subcore_axis_name="s")

  @functools.partial(
      pl.kernel,
      out_type=[
          jax.ShapeDtypeStruct((_B, _D), jnp.float32),
          jax.ShapeDtypeStruct((_B, _D), jnp.float32),
          jax.ShapeDtypeStruct((_B * _NS, _D), jnp.float32),
      ],
      mesh=mesh,
      scratch_types=(
          [pltpu.VMEM((nch_noise, _C), jnp.int32),
           pltpu.VMEM((nch_noise, _C), jnp.int32),
           pltpu.VMEM((n_small // _C, _C), jnp.int32)]
          + [pltpu.VMEM((_C, _D), jnp.float32) for _ in range(_NBUF)]
          + [pltpu.SemaphoreType.DMA for _ in range(2 * _NBUF + 1)]
      ),
  )
  def gather_kernel(iw_hbm, ow_hbm, nw_hbm, ie_hbm, oe_hbm, o1, o2, o3, *scr):
    idxrows = scr[0]
    outrows = scr[1]
    sidx = scr[2]
    rows = scr[3:3 + _NBUF]
    gsem = scr[3 + _NBUF:3 + 2 * _NBUF]
    wsem = scr[3 + 2 * _NBUF:3 + 3 * _NBUF]
    nsem = scr[3 + 3 * _NBUF]
    wid = lax.axis_index("s") * nc + lax.axis_index("c")
    lanes = lax.iota(jnp.int32, _L)

    pltpu.async_copy(nw_hbm.at[pl.ds(wid * nch_noise, nch_noise)], idxrows,
                     nsem)

    def small_phase(idx2_hbm, tab_hbm, out_hbm):
      nch = n_small // _C
      base = wid * n_small
      pltpu.sync_copy(idx2_hbm.at[pl.ds(wid * nch, nch)], sidx)
      for b in range(nch):
        pltpu.async_copy(tab_hbm.at[sidx.at[b]], rows[b], gsem[b])
      for b in range(nch):
        out_slice = out_hbm.at[pl.ds(base + b * _C, _C)]
        pltpu.make_async_copy(tab_hbm.at[sidx.at[b]], rows[b], gsem[b]).wait()
        pltpu.async_copy(rows[b], out_slice, wsem[b])
      for b in range(nch):
        pltpu.make_async_copy(rows[b], out_hbm.at[pl.ds(base, _C)], wsem[b]).wait()

    small_phase(iw_hbm, ie_hbm, o1)
    small_phase(ow_hbm, oe_hbm, o2)

    def orow(r, carry):
      nvec, bvec = carry
      for j in range(_C // _L):
        outrows[r, pl.ds(j * _L, _L)] = nvec * _B + bvec
        nxt = nvec + _L
        wrap = nxt >= _NS
        nvec = jnp.where(wrap, nxt - _NS, nxt)
        bvec = jnp.where(wrap, bvec + 1, bvec)
      return (nvec, bvec)

    lax.fori_loop(0, nch_noise, orow,
                  (lanes, jnp.full((_L,), wid * (_B // nw), jnp.int32)))

    pltpu.make_async_copy(nw_hbm.at[pl.ds(wid * nch_noise, nch_noise)],
                          idxrows, nsem).wait()
    for b in range(_NBUF):
      pltpu.async_copy(oe_hbm.at[idxrows.at[b]], rows[b], gsem[b])

    def step(s, carry):
      for b in range(_NBUF):
        g = s * _NBUF + b
        pltpu.make_async_copy(oe_hbm.at[idxrows.at[g]], rows[b], gsem[b]).wait()
        pltpu.async_copy(rows[b], o3.at[outrows.at[g]], wsem[b])

        @pl.when(s < (nch_noise // _NBUF) - 1)
        def _():
          pltpu.make_async_copy(rows[b], o3.at[outrows.at[g]], wsem[b]).wait()
          pltpu.async_copy(oe_hbm.at[idxrows.at[g + _NBUF]], rows[b], gsem[b])

      return carry

    lax.fori_loop(0, nch_noise // _NBUF, step, 0)
    for b in range(_NBUF):
      pltpu.make_async_copy(rows[b], o3.at[outrows.at[nch_noise - _NBUF + b]],
                            wsem[b]).wait()

  return gather_kernel


def kernel(input_words, output_words, noise_words, in_embeddings, out_embeddings):
  gather = _build_gather_kernel()
  o1, o2, o3 = gather(
      input_words.astype(jnp.int32).reshape(-1, _C),
      output_words.astype(jnp.int32).reshape(-1, _C),
      noise_words.astype(jnp.int32).reshape(-1, _C),
      in_embeddings,
      out_embeddings,
  )
  return (o1, o2, jnp.swapaxes(o3.reshape(_NS, _B, _D), 0, 1))
